# baseline (speedup 1.0000x reference)
import functools

import jax
import jax.numpy as jnp
from jax import lax
from jax.experimental import pallas as pl
from jax.experimental.pallas import tpu as pltpu
from jax.experimental.pallas import tpu_sc as plsc

_B, _S, _D, _V, _DENC = 4, 8192, 1024, 32000, 512
_NIDS = _B * _S
_CHUNKS = (4096,) * 8
_BLK = 512
_NW = 32
_CH = 32


@functools.cache
def _make_gather(crows):
    bpw = crows // _NW
    nch = bpw // _CH

    @functools.partial(
        pl.kernel,
        out_type=jax.ShapeDtypeStruct((crows, _D), jnp.float32),
        mesh=plsc.VectorSubcoreMesh(core_axis_name="c", subcore_axis_name="s"),
        scratch_types=[
            pltpu.VMEM((bpw,), jnp.int32),
            pltpu.VMEM((_CH, _D), jnp.float32),
            pltpu.VMEM((_CH, _D), jnp.float32),
            pltpu.SemaphoreType.DMA,
            pltpu.SemaphoreType.DMA,
            pltpu.SemaphoreType.DMA,
            pltpu.SemaphoreType.DMA,
        ],
    )
    def _gather_k(t_hbm, i_hbm, o_hbm, idx_v, r0, r1, gs0, gs1, ss0, ss1):
        wid = lax.axis_index("s") * 2 + lax.axis_index("c")
        base = wid * bpw
        pltpu.sync_copy(i_hbm.at[pl.ds(base, bpw)], idx_v)

        def gcopy(chunk, buf, sem):
            return pltpu.make_async_copy(
                t_hbm.at[idx_v.at[pl.ds(chunk * _CH, _CH)]], buf, sem)

        def scopy(chunk, buf, sem):
            return pltpu.make_async_copy(
                buf, o_hbm.at[pl.ds(base + chunk * _CH, _CH)], sem)

        gcopy(0, r0, gs0).start()
        gcopy(1, r1, gs1).start()

        @pl.loop(0, nch, step=2)
        def _(g):
            gcopy(g, r0, gs0).wait()
            scopy(g, r0, ss0).start()
            gcopy(g + 1, r1, gs1).wait()
            scopy(g + 1, r1, ss1).start()

            @pl.when(g + 2 < nch)
            def _():
                scopy(g, r0, ss0).wait()
                gcopy(g + 2, r0, gs0).start()
                scopy(g + 1, r1, ss1).wait()
                gcopy(g + 3, r1, gs1).start()

        scopy(nch - 2, r0, ss0).wait()
        scopy(nch - 1, r1, ss1).wait()

    return _gather_k


def _mm_chunk(start, e_k, pos, sf_bf, wenc_bf, wlm_bf, out_prev):
    crows = e_k.shape[0]

    def body(*refs):
        pos_ref, sf_ref, wenc_ref, wlm_ref, e_ref = refs[:5]
        o_ref = refs[-1]
        i = pl.program_id(0)
        r0 = start + i * _BLK
        wlm = wlm_ref[...]
        o_ref[...] = jnp.dot(e_ref[...].astype(jnp.bfloat16), wlm,
                             preferred_element_type=jnp.float32)
        enc = jnp.dot(sf_ref[...], wenc_ref[...],
                      preferred_element_type=jnp.float32)
        encp = jnp.dot(enc.astype(jnp.bfloat16), wlm,
                       preferred_element_type=jnp.float32)
        for b in range(_B):
            p = pos_ref[b]

            @pl.when((p >= r0) & (p < r0 + _BLK))
            def _():
                o_ref[pl.ds(p - r0, 1), :] = encp[b:b + 1, :]

    nsteps = crows // _BLK
    base_blk = start // _BLK
    in_specs = [
        pl.BlockSpec(memory_space=pltpu.MemorySpace.SMEM),
        pl.BlockSpec((_B, _DENC), lambda i: (0, 0)),
        pl.BlockSpec((_DENC, _D), lambda i: (0, 0)),
        pl.BlockSpec((_D, _D), lambda i: (0, 0)),
        pl.BlockSpec((_BLK, _D), lambda i: (i, 0)),
    ]
    args = [pos, sf_bf, wenc_bf, wlm_bf, e_k]
    aliases = {}
    if out_prev is not None:
        in_specs.append(pl.BlockSpec(memory_space=pl.ANY))
        args.append(out_prev)
        aliases = {5: 0}
    return pl.pallas_call(
        body,
        grid=(nsteps,),
        in_specs=in_specs,
        out_specs=pl.BlockSpec((_BLK, _D), lambda i: (base_blk + i, 0)),
        out_shape=jax.ShapeDtypeStruct((_NIDS, _D), jnp.float32),
        input_output_aliases=aliases,
    )(*args)


def kernel(elm_input_ids, signal_id_indices, signal_feats, embed_table,
           W_enc, W_lm):
    ids = elm_input_ids.reshape(_NIDS)
    pos = (jnp.arange(_B, dtype=jnp.int32) * _S
           + signal_id_indices.astype(jnp.int32))
    sf_bf = signal_feats.astype(jnp.bfloat16)
    wenc_bf = W_enc.astype(jnp.bfloat16)
    wlm_bf = W_lm.astype(jnp.bfloat16)

    out = None
    start = 0
    for crows in _CHUNKS:
        e_k = _make_gather(crows)(
            embed_table, lax.slice(ids, (start,), (start + crows,)))
        out = _mm_chunk(start, e_k, pos, sf_bf, wenc_bf, wlm_bf, out)
        start += crows
    return out.reshape(_B, _S, _D)


# speedup vs baseline: 1.0812x; 1.0812x over previous
import functools

import jax
import jax.numpy as jnp
from jax import lax
from jax.experimental import pallas as pl
from jax.experimental.pallas import tpu as pltpu
from jax.experimental.pallas import tpu_sc as plsc

_B, _S, _D, _V, _DENC = 4, 8192, 1024, 32000, 512
_NIDS = _B * _S
_CHUNKS = (4096,) * 8
_BLK = 1024
_NW = 32
_CH = 32


@functools.cache
def _make_gather(crows):
    bpw = crows // _NW
    nch = bpw // _CH

    @functools.partial(
        pl.kernel,
        out_type=jax.ShapeDtypeStruct((crows, _D), jnp.float32),
        mesh=plsc.VectorSubcoreMesh(core_axis_name="c", subcore_axis_name="s"),
        scratch_types=[
            pltpu.VMEM((bpw,), jnp.int32),
            pltpu.VMEM((_CH, _D), jnp.float32),
            pltpu.VMEM((_CH, _D), jnp.float32),
            pltpu.SemaphoreType.DMA,
            pltpu.SemaphoreType.DMA,
        ],
    )
    def _gather_k(t_hbm, i_hbm, o_hbm, idx_v, r0, r1, gs0, gs1):
        wid = lax.axis_index("s") * 2 + lax.axis_index("c")
        base = wid * bpw
        pltpu.sync_copy(i_hbm.at[pl.ds(base, bpw)], idx_v)

        def gcopy(chunk, buf, sem):
            return pltpu.make_async_copy(
                t_hbm.at[idx_v.at[pl.ds(chunk * _CH, _CH)]], buf, sem)

        def store(chunk, buf):
            pltpu.sync_copy(buf, o_hbm.at[pl.ds(base + chunk * _CH, _CH)])

        gcopy(0, r0, gs0).start()

        @pl.loop(0, nch, step=2)
        def _(g):
            gcopy(g + 1, r1, gs1).start()
            gcopy(g, r0, gs0).wait()
            store(g, r0)

            @pl.when(g + 2 < nch)
            def _():
                gcopy(g + 2, r0, gs0).start()

            gcopy(g + 1, r1, gs1).wait()
            store(g + 1, r1)

    return _gather_k


def _mm_chunk(start, e_k, pos, sf_bf, wenc_bf, wlm_bf, out_prev):
    crows = e_k.shape[0]

    def body(*refs):
        pos_ref, sf_ref, wenc_ref, wlm_ref, e_ref = refs[:5]
        o_ref = refs[-1]
        i = pl.program_id(0)
        r0 = start + i * _BLK
        wlm = wlm_ref[...]
        o_ref[...] = jnp.dot(e_ref[...].astype(jnp.bfloat16), wlm,
                             preferred_element_type=jnp.float32)
        enc = jnp.dot(sf_ref[...], wenc_ref[...],
                      preferred_element_type=jnp.float32)
        encp = jnp.dot(enc.astype(jnp.bfloat16), wlm,
                       preferred_element_type=jnp.float32)
        for b in range(_B):
            p = pos_ref[b]

            @pl.when((p >= r0) & (p < r0 + _BLK))
            def _():
                o_ref[pl.ds(p - r0, 1), :] = encp[b:b + 1, :]

    nsteps = crows // _BLK
    base_blk = start // _BLK
    in_specs = [
        pl.BlockSpec(memory_space=pltpu.MemorySpace.SMEM),
        pl.BlockSpec((_B, _DENC), lambda i: (0, 0)),
        pl.BlockSpec((_DENC, _D), lambda i: (0, 0)),
        pl.BlockSpec((_D, _D), lambda i: (0, 0)),
        pl.BlockSpec((_BLK, _D), lambda i: (i, 0)),
    ]
    args = [pos, sf_bf, wenc_bf, wlm_bf, e_k]
    aliases = {}
    if out_prev is not None:
        in_specs.append(pl.BlockSpec(memory_space=pl.ANY))
        args.append(out_prev)
        aliases = {5: 0}
    return pl.pallas_call(
        body,
        grid=(nsteps,),
        in_specs=in_specs,
        out_specs=pl.BlockSpec((_BLK, _D), lambda i: (base_blk + i, 0)),
        out_shape=jax.ShapeDtypeStruct((_NIDS, _D), jnp.float32),
        input_output_aliases=aliases,
    )(*args)


def kernel(elm_input_ids, signal_id_indices, signal_feats, embed_table,
           W_enc, W_lm):
    ids = elm_input_ids.reshape(_NIDS)
    pos = (jnp.arange(_B, dtype=jnp.int32) * _S
           + signal_id_indices.astype(jnp.int32))
    sf_bf = signal_feats.astype(jnp.bfloat16)
    wenc_bf = W_enc.astype(jnp.bfloat16)
    wlm_bf = W_lm.astype(jnp.bfloat16)

    out = None
    start = 0
    for crows in _CHUNKS:
        e_k = _make_gather(crows)(
            embed_table, lax.slice(ids, (start,), (start + crows,)))
        out = _mm_chunk(start, e_k, pos, sf_bf, wenc_bf, wlm_bf, out)
        start += crows
    return out.reshape(_B, _S, _D)
